# baseline (speedup 1.0000x reference)
import jax
import jax.numpy as jnp
from jax.experimental import pallas as pl
from jax.experimental.pallas import tpu as pltpu


def _fused_body(z_ref, m_ref, win_ref, bin_ref, wout_ref, bout_ref,
                out_ref, h_ref):
    h = jnp.dot(z_ref[...].astype(jnp.bfloat16), win_ref[...].astype(jnp.bfloat16),
                preferred_element_type=jnp.float32) + bin_ref[...]
    h_ref[...] = h
    o = jnp.dot(h.astype(jnp.bfloat16), wout_ref[...].astype(jnp.bfloat16),
                preferred_element_type=jnp.float32) + bout_ref[...]
    out_ref[...] = o * m_ref[...]


def kernel(z, mask, W_in, b_in, W_out, b_out):
    B, L, D = z.shape
    C = W_in.shape[1]
    N = B * L
    BLK = 2048

    zf = z.reshape(N, D)
    mf = mask.reshape(N, 1).astype(jnp.float32)
    bin2 = b_in.reshape(1, C)
    bout2 = b_out.reshape(1, D)

    out, hid = pl.pallas_call(
        _fused_body,
        grid=(N // BLK,),
        in_specs=[
            pl.BlockSpec((BLK, D), lambda i: (i, 0)),
            pl.BlockSpec((BLK, 1), lambda i: (i, 0)),
            pl.BlockSpec((D, C), lambda i: (0, 0)),
            pl.BlockSpec((1, C), lambda i: (0, 0)),
            pl.BlockSpec((C, D), lambda i: (0, 0)),
            pl.BlockSpec((1, D), lambda i: (0, 0)),
        ],
        out_specs=[
            pl.BlockSpec((BLK, D), lambda i: (i, 0)),
            pl.BlockSpec((BLK, C), lambda i: (i, 0)),
        ],
        out_shape=[
            jax.ShapeDtypeStruct((N, D), jnp.float32),
            jax.ShapeDtypeStruct((N, C), jnp.float32),
        ],
        compiler_params=pltpu.CompilerParams(
            dimension_semantics=("parallel",),
        ),
    )(zf, mf, W_in, bin2, W_out, bout2)

    return out.reshape(B, L, D), hid.reshape(B, L, C)


# speedup vs baseline: 1.0026x; 1.0026x over previous
import jax
import jax.numpy as jnp
from jax.experimental import pallas as pl
from jax.experimental.pallas import tpu as pltpu


def _fused_body(z_ref, m_ref, win_ref, bin_ref, wout_ref, bout_ref,
                out_ref, h_ref):
    h = jnp.dot(z_ref[...], win_ref[...],
                preferred_element_type=jnp.float32) + bin_ref[...]
    h_ref[...] = h
    o = jnp.dot(h, wout_ref[...],
                preferred_element_type=jnp.float32) + bout_ref[...]
    out_ref[...] = o * m_ref[...]


def kernel(z, mask, W_in, b_in, W_out, b_out):
    B, L, D = z.shape
    C = W_in.shape[1]
    N = B * L
    BLK = 2048

    zf = z.reshape(N, D)
    mf = mask.reshape(N, 1).astype(jnp.float32)
    bin2 = b_in.reshape(1, C)
    bout2 = b_out.reshape(1, D)

    out, hid = pl.pallas_call(
        _fused_body,
        grid=(N // BLK,),
        in_specs=[
            pl.BlockSpec((BLK, D), lambda i: (i, 0)),
            pl.BlockSpec((BLK, 1), lambda i: (i, 0)),
            pl.BlockSpec((D, C), lambda i: (0, 0)),
            pl.BlockSpec((1, C), lambda i: (0, 0)),
            pl.BlockSpec((C, D), lambda i: (0, 0)),
            pl.BlockSpec((1, D), lambda i: (0, 0)),
        ],
        out_specs=[
            pl.BlockSpec((BLK, D), lambda i: (i, 0)),
            pl.BlockSpec((BLK, C), lambda i: (i, 0)),
        ],
        out_shape=[
            jax.ShapeDtypeStruct((N, D), jnp.float32),
            jax.ShapeDtypeStruct((N, C), jnp.float32),
        ],
        compiler_params=pltpu.CompilerParams(
            dimension_semantics=("parallel",),
        ),
    )(zf, mf, W_in, bin2, W_out, bout2)

    return out.reshape(B, L, D), hid.reshape(B, L, C)
